# Initial kernel scaffold; baseline (speedup 1.0000x reference)
import jax
import jax.numpy as jnp
from jax.experimental import pallas as pl


def kernel(h, edge_index, coord, edge_attr, W_e1, b_e1, W_e2, b_e2, W_n1, b_n1, W_n2, b_n2, W_c1, b_c1, W_c2):
    raise NotImplementedError("write your pallas kernel here")


# speedup vs baseline: 2.4617x; 2.4617x over previous
import functools

import jax
import jax.numpy as jnp
from jax import lax
from jax.experimental import pallas as pl
from jax.experimental.pallas import tpu as pltpu
from jax.experimental.pallas import tpu_sc as plsc

F32 = jnp.float32

SC_CORES = 2
SC_SUBCORES = 16
SC_WORKERS = SC_CORES * SC_SUBCORES

TW = 256
GW = 128
SC_CHUNK = 128
NPAD = 10240
BN = 400
BE = 512


def _silu(x):
    return x / (1.0 + jnp.exp(-x))


def _k1_body(h_ref, cpad_ref, w1s_ref, w1d_ref, be1_ref, tsrc_ref, tdst_ref):
    hb = h_ref[...]
    cp = cpad_ref[...]
    a_s = jnp.dot(hb, w1s_ref[...], preferred_element_type=F32) + be1_ref[...]
    a_d = jnp.dot(hb, w1d_ref[...], preferred_element_type=F32)
    tsrc_ref[...] = jnp.concatenate([a_s, cp], axis=1)
    tdst_ref[...] = jnp.concatenate([a_d, cp], axis=1)


def _build_tables(h, coord_pad, w1s, w1d, be1):
    n, d = h.shape
    return pl.pallas_call(
        _k1_body,
        grid=(n // BN,),
        in_specs=[
            pl.BlockSpec((BN, d), lambda i: (i, 0)),
            pl.BlockSpec((BN, 128), lambda i: (i, 0)),
            pl.BlockSpec((d, 128), lambda i: (0, 0)),
            pl.BlockSpec((d, 128), lambda i: (0, 0)),
            pl.BlockSpec((1, 128), lambda i: (0, 0)),
        ],
        out_specs=[
            pl.BlockSpec((BN, TW), lambda i: (i, 0)),
            pl.BlockSpec((BN, TW), lambda i: (i, 0)),
        ],
        out_shape=[
            jax.ShapeDtypeStruct((n, TW), F32),
            jax.ShapeDtypeStruct((n, TW), F32),
        ],
    )(h, coord_pad, w1s, w1d, be1)


def _sc_gather(table, idx2d, e):
    mesh = plsc.VectorSubcoreMesh(
        core_axis_name="c", subcore_axis_name="s",
        num_cores=SC_CORES, num_subcores=SC_SUBCORES)

    @functools.partial(
        pl.kernel,
        out_type=jax.ShapeDtypeStruct((e, TW), F32),
        mesh=mesh)
    def gk(table_hbm, idx_hbm, out_hbm):
        def body(i_vmem, o_vmem):
            pltpu.sync_copy(table_hbm.at[i_vmem.at[0]], o_vmem)

        pltpu.emit_pipeline(
            body,
            grid=(e // GW,),
            in_specs=[pl.BlockSpec((1, GW), lambda i: (0, i))],
            out_specs=[pl.BlockSpec((GW, TW), lambda i: (i, 0))],
            core_axis_name=("c", "s"),
            dimension_semantics=(pltpu.PARALLEL,),
        )(idx_hbm, out_hbm)

    return gk(table, idx2d)


def _k3_body(gs_ref, gd_ref, ea_ref, wea_ref, wr_ref,
             we2_ref, be2_ref, wc1_ref, bc1_ref, wc2t_ref, ef_ref, aux_ref):
    gs = gs_ref[...]
    gd = gd_ref[...]
    a = gs[:, :128] + gd[:, :128]
    cd = gs[:, 128:131] - gd[:, 128:131]
    radial = jnp.sum(cd * cd, axis=1, keepdims=True)
    pre = a + radial * wr_ref[...] + jnp.dot(
        ea_ref[...], wea_ref[...], preferred_element_type=F32)
    m = _silu(pre)
    ef = _silu(jnp.dot(m, we2_ref[...], preferred_element_type=F32)
               + be2_ref[...])
    c = _silu(jnp.dot(ef, wc1_ref[...], preferred_element_type=F32)
              + bc1_ref[...])
    s = jnp.sum(c * wc2t_ref[...], axis=1, keepdims=True)
    trans = cd * s
    ef_ref[...] = ef
    aux_ref[:, 0:3] = trans
    aux_ref[:, 3:4] = jnp.ones((trans.shape[0], 1), F32)
    aux_ref[:, 4:128] = jnp.zeros((trans.shape[0], 124), F32)


def _edge_compute(gs, gd, edge_attr, wea, wr, we2, be2, wc1, bc1, wc2t):
    e = edge_attr.shape[0]
    return pl.pallas_call(
        _k3_body,
        grid=(e // BE,),
        in_specs=[
            pl.BlockSpec((BE, TW), lambda i: (i, 0)),
            pl.BlockSpec((BE, TW), lambda i: (i, 0)),
            pl.BlockSpec((BE, 16), lambda i: (i, 0)),
            pl.BlockSpec((16, 128), lambda i: (0, 0)),
            pl.BlockSpec((1, 128), lambda i: (0, 0)),
            pl.BlockSpec((128, 128), lambda i: (0, 0)),
            pl.BlockSpec((1, 128), lambda i: (0, 0)),
            pl.BlockSpec((128, 128), lambda i: (0, 0)),
            pl.BlockSpec((1, 128), lambda i: (0, 0)),
            pl.BlockSpec((1, 128), lambda i: (0, 0)),
        ],
        out_specs=[
            pl.BlockSpec((BE, 128), lambda i: (i, 0)),
            pl.BlockSpec((BE, 128), lambda i: (i, 0)),
        ],
        out_shape=[
            jax.ShapeDtypeStruct((e, 128), F32),
            jax.ShapeDtypeStruct((e, 128), F32),
        ],
    )(gs, gd, edge_attr, wea, wr, we2, be2, wc1, bc1, wc2t)


def _sc_scatter(rowp, ef, aux, z128, npad, e, epad):
    mesh = plsc.VectorSubcoreMesh(
        core_axis_name="c", subcore_axis_name="s",
        num_cores=SC_CORES, num_subcores=SC_SUBCORES)
    chunks_per_w = epad // SC_CHUNK // SC_WORKERS
    data_chunks = e // SC_CHUNK
    n_per_sub = npad // SC_SUBCORES

    @functools.partial(
        pl.kernel,
        out_type=(jax.ShapeDtypeStruct((SC_CORES * npad, 128), F32),
                  jax.ShapeDtypeStruct((SC_CORES * npad, 128), F32)),
        mesh=mesh,
        scratch_types=[
            pltpu.VMEM((SC_CHUNK,), jnp.int32),
            pltpu.VMEM((SC_CHUNK, 128), F32),
            pltpu.VMEM_SHARED((npad, 128), F32),
        ])
    def sk(row_hbm, ef_hbm, aux_hbm, z128_hbm, oagg_hbm, oaux_hbm,
           idx_v, buf_v, sh):
        c = lax.axis_index("c")
        s = lax.axis_index("s")
        wid = c * SC_SUBCORES + s
        nb = pl.multiple_of(s * n_per_sub, 8)
        n_steps = n_per_sub // SC_CHUNK

        def phase(data_hbm, out_hbm):
            pltpu.sync_copy(z128_hbm, buf_v)

            @pl.loop(0, n_steps)
            def _(j):
                o = pl.multiple_of(nb + j * SC_CHUNK, 8)
                pltpu.sync_copy(buf_v, sh.at[pl.ds(o, SC_CHUNK)])

            plsc.subcore_barrier()

            @pl.loop(0, chunks_per_w)
            def _(j):
                ch = wid * chunks_per_w + j
                off_i = pl.multiple_of(ch * SC_CHUNK, SC_CHUNK)
                off_d = pl.multiple_of(
                    jnp.where(ch < data_chunks, ch, 0) * SC_CHUNK, SC_CHUNK)
                pltpu.sync_copy(row_hbm.at[pl.ds(off_i, SC_CHUNK)], idx_v)
                pltpu.sync_copy(data_hbm.at[pl.ds(off_d, SC_CHUNK)], buf_v)
                pltpu.sync_copy(buf_v, sh.at[idx_v], add=True)

            plsc.subcore_barrier()

            @pl.loop(0, n_steps)
            def _(j):
                o = pl.multiple_of(nb + j * SC_CHUNK, 8)
                oo = pl.multiple_of(c * npad + o, 8)
                pltpu.sync_copy(sh.at[pl.ds(o, SC_CHUNK)], buf_v)
                pltpu.sync_copy(buf_v, out_hbm.at[pl.ds(oo, SC_CHUNK)])

        phase(ef_hbm, oagg_hbm)
        phase(aux_hbm, oaux_hbm)

    return sk(rowp, ef, aux, z128)


def _k5_body(h_ref, cpad_ref, agg0_ref, agg1_ref, aux0_ref, aux1_ref,
             wn1a_ref, wn1b_ref, bn1_ref, wn2_ref, bn2_ref,
             hout_ref, cout_ref):
    agg = agg0_ref[0] + agg1_ref[0]
    aux = aux0_ref[0] + aux1_ref[0]
    seg = aux[:, 0:3]
    cnt = aux[:, 3:4]
    cout_ref[...] = cpad_ref[...][:, 0:3] + seg / jnp.maximum(cnt, 1.0)
    hb = h_ref[...]
    t = _silu(jnp.dot(hb, wn1a_ref[...], preferred_element_type=F32)
              + jnp.dot(agg, wn1b_ref[...], preferred_element_type=F32)
              + bn1_ref[...])
    hout_ref[...] = hb + jnp.dot(t, wn2_ref[...], preferred_element_type=F32) \
        + bn2_ref[...]


def _node_update(h, coord_pad, aggp, auxp, wn1a, wn1b, bn1, wn2, bn2):
    n, d = h.shape
    return pl.pallas_call(
        _k5_body,
        grid=(n // BN,),
        in_specs=[
            pl.BlockSpec((BN, d), lambda i: (i, 0)),
            pl.BlockSpec((BN, 128), lambda i: (i, 0)),
            pl.BlockSpec((1, BN, 128), lambda i: (0, i, 0)),
            pl.BlockSpec((1, BN, 128), lambda i: (1, i, 0)),
            pl.BlockSpec((1, BN, 128), lambda i: (0, i, 0)),
            pl.BlockSpec((1, BN, 128), lambda i: (1, i, 0)),
            pl.BlockSpec((d, 128), lambda i: (0, 0)),
            pl.BlockSpec((128, 128), lambda i: (0, 0)),
            pl.BlockSpec((1, 128), lambda i: (0, 0)),
            pl.BlockSpec((128, d), lambda i: (0, 0)),
            pl.BlockSpec((1, d), lambda i: (0, 0)),
        ],
        out_specs=[
            pl.BlockSpec((BN, d), lambda i: (i, 0)),
            pl.BlockSpec((BN, 3), lambda i: (i, 0)),
        ],
        out_shape=[
            jax.ShapeDtypeStruct((n, d), F32),
            jax.ShapeDtypeStruct((n, 3), F32),
        ],
    )(h, coord_pad, aggp, aggp, auxp, auxp, wn1a, wn1b, bn1, wn2, bn2)


def kernel(h, edge_index, coord, edge_attr,
           W_e1, b_e1, W_e2, b_e2,
           W_n1, b_n1, W_n2, b_n2,
           W_c1, b_c1, W_c2):
    n, d = h.shape
    e = edge_index.shape[1]
    row = edge_index[0]
    col = edge_index[1]

    coord_pad = jnp.pad(coord, ((0, 0), (0, 125)))
    w1s = W_e1[:d]
    w1d = W_e1[d:2 * d]
    wr = W_e1[2 * d:2 * d + 1]
    wea = W_e1[2 * d + 1:]
    be1 = b_e1.reshape(1, -1)
    be2 = b_e2.reshape(1, -1)
    bc1 = b_c1.reshape(1, -1)
    bn1 = b_n1.reshape(1, -1)
    bn2 = b_n2.reshape(1, -1)
    wc2t = W_c2.reshape(1, -1)
    wn1a = W_n1[:d]
    wn1b = W_n1[d:]

    tsrc, tdst = _build_tables(h, coord_pad, w1s, w1d, be1)
    epad = -(-e // (GW * SC_WORKERS)) * (GW * SC_WORKERS)
    rowp = jnp.pad(row, (0, epad - e)).reshape(1, epad)
    colp = jnp.pad(col, (0, epad - e)).reshape(1, epad)
    gs = _sc_gather(tsrc, rowp, epad)
    gd = _sc_gather(tdst, colp, epad)
    ef, aux = _edge_compute(gs, gd, edge_attr, wea, wr, we2=W_e2,
                            be2=be2, wc1=W_c1, bc1=bc1, wc2t=wc2t)
    z128 = jnp.zeros((SC_CHUNK, 128), F32)
    row_scat = jnp.concatenate([row, jnp.full((epad - e,), n, jnp.int32)])
    aggp, auxp = _sc_scatter(row_scat, ef, aux, z128, NPAD, e, epad)
    aggp = aggp.reshape(SC_CORES, NPAD, 128)
    auxp = auxp.reshape(SC_CORES, NPAD, 128)
    h_out, coord_out = _node_update(h, coord_pad, aggp, auxp,
                                    wn1a, wn1b, bn1, W_n2, bn2)
    return (h_out, coord_out, edge_attr, ef)


# speedup vs baseline: 2.8957x; 1.1763x over previous
import functools

import numpy as np

import jax
import jax.numpy as jnp
from jax import lax
from jax.experimental import pallas as pl
from jax.experimental.pallas import tpu as pltpu
from jax.experimental.pallas import tpu_sc as plsc

F32 = jnp.float32
BF16 = jnp.bfloat16

SC_CORES = 2
SC_SUBCORES = 16
SC_WORKERS = SC_CORES * SC_SUBCORES

TW = 256
GW = 128
SC_CHUNK = 128
NPAD = 10240
BN = 400
BE = 512


def _silu(x):
    return x / (1.0 + jnp.exp(-x))


_HI = np.uint32(0xFFFF0000)
_RND = np.uint32(0x8000)


def _pack_hi(x):
    return (lax.bitcast_convert_type(x, jnp.uint32) + _RND) & _HI


def _k1_body(h_ref, cpad_ref, w1s_ref, w1d_ref, be1_ref, tsrc_ref, tdst_ref):
    hb = h_ref[...]
    lo = (lax.bitcast_convert_type(cpad_ref[...], jnp.uint32) + _RND) >> 16
    a_s = jnp.dot(hb, w1s_ref[...], preferred_element_type=F32) + be1_ref[...]
    a_d = jnp.dot(hb, w1d_ref[...], preferred_element_type=F32)
    tsrc_ref[...] = lax.bitcast_convert_type(_pack_hi(a_s) | lo, F32)
    tdst_ref[...] = lax.bitcast_convert_type(_pack_hi(a_d) | lo, F32)


def _build_tables(h, coord_pad, w1s, w1d, be1):
    n, d = h.shape
    return pl.pallas_call(
        _k1_body,
        grid=(n // BN,),
        in_specs=[
            pl.BlockSpec((BN, d), lambda i: (i, 0)),
            pl.BlockSpec((BN, 128), lambda i: (i, 0)),
            pl.BlockSpec((d, 128), lambda i: (0, 0)),
            pl.BlockSpec((d, 128), lambda i: (0, 0)),
            pl.BlockSpec((1, 128), lambda i: (0, 0)),
        ],
        out_specs=[
            pl.BlockSpec((BN, 128), lambda i: (i, 0)),
            pl.BlockSpec((BN, 128), lambda i: (i, 0)),
        ],
        out_shape=[
            jax.ShapeDtypeStruct((n, 128), F32),
            jax.ShapeDtypeStruct((n, 128), F32),
        ],
    )(h, coord_pad, w1s, w1d, be1)


def _sc_gather(table, idx2d, e):
    mesh = plsc.VectorSubcoreMesh(
        core_axis_name="c", subcore_axis_name="s",
        num_cores=SC_CORES, num_subcores=SC_SUBCORES)

    @functools.partial(
        pl.kernel,
        out_type=jax.ShapeDtypeStruct((e, 128), F32),
        mesh=mesh)
    def gk(table_hbm, idx_hbm, out_hbm):
        def body(i_vmem, o_vmem):
            pltpu.sync_copy(table_hbm.at[i_vmem.at[0]], o_vmem)

        pltpu.emit_pipeline(
            body,
            grid=(e // GW,),
            in_specs=[pl.BlockSpec((1, GW), lambda i: (0, i))],
            out_specs=[pl.BlockSpec((GW, 128), lambda i: (i, 0))],
            core_axis_name=("c", "s"),
            dimension_semantics=(pltpu.PARALLEL,),
        )(idx_hbm, out_hbm)

    return gk(table, idx2d)


def _k3_body(gs_ref, gd_ref, ea_ref, wea_ref, wr_ref,
             we2_ref, be2_ref, wc1_ref, bc1_ref, wc2t_ref, ef_ref, aux_ref):
    w_s = lax.bitcast_convert_type(gs_ref[...], jnp.uint32)
    w_d = lax.bitcast_convert_type(gd_ref[...], jnp.uint32)
    a = (lax.bitcast_convert_type(w_s & _HI, F32)
         + lax.bitcast_convert_type(w_d & _HI, F32))
    cs = lax.bitcast_convert_type(w_s << 16, F32)
    cdd = lax.bitcast_convert_type(w_d << 16, F32)
    cd = cs[:, 0:3] - cdd[:, 0:3]
    radial = jnp.sum(cd * cd, axis=1, keepdims=True)
    pre = a + radial * wr_ref[...] + jnp.dot(
        ea_ref[...], wea_ref[...], preferred_element_type=F32)
    m = _silu(pre)
    ef = _silu(jnp.dot(m, we2_ref[...], preferred_element_type=F32)
               + be2_ref[...])
    c = _silu(jnp.dot(ef, wc1_ref[...], preferred_element_type=F32)
              + bc1_ref[...])
    s = jnp.sum(c * wc2t_ref[...], axis=1, keepdims=True)
    trans = cd * s
    ef_ref[...] = ef
    aux_ref[:, 0:3] = trans
    aux_ref[:, 3:4] = jnp.ones((trans.shape[0], 1), F32)
    aux_ref[:, 4:128] = jnp.zeros((trans.shape[0], 124), F32)


def _edge_compute(gs, gd, edge_attr, wea, wr, we2, be2, wc1, bc1, wc2t):
    e = edge_attr.shape[0]
    return pl.pallas_call(
        _k3_body,
        grid=(e // BE,),
        in_specs=[
            pl.BlockSpec((BE, 128), lambda i: (i, 0)),
            pl.BlockSpec((BE, 128), lambda i: (i, 0)),
            pl.BlockSpec((BE, 16), lambda i: (i, 0)),
            pl.BlockSpec((16, 128), lambda i: (0, 0)),
            pl.BlockSpec((1, 128), lambda i: (0, 0)),
            pl.BlockSpec((128, 128), lambda i: (0, 0)),
            pl.BlockSpec((1, 128), lambda i: (0, 0)),
            pl.BlockSpec((128, 128), lambda i: (0, 0)),
            pl.BlockSpec((1, 128), lambda i: (0, 0)),
            pl.BlockSpec((1, 128), lambda i: (0, 0)),
        ],
        out_specs=[
            pl.BlockSpec((BE, 128), lambda i: (i, 0)),
            pl.BlockSpec((BE, 128), lambda i: (i, 0)),
        ],
        out_shape=[
            jax.ShapeDtypeStruct((e, 128), F32),
            jax.ShapeDtypeStruct((e, 128), F32),
        ],
    )(gs, gd, edge_attr, wea, wr, we2, be2, wc1, bc1, wc2t)


def _sc_scatter(rowp, ef, aux, z128, npad, e, epad):
    mesh = plsc.VectorSubcoreMesh(
        core_axis_name="c", subcore_axis_name="s",
        num_cores=SC_CORES, num_subcores=SC_SUBCORES)
    chunks_per_w = epad // SC_CHUNK // SC_WORKERS
    data_chunks = e // SC_CHUNK
    n_per_sub = npad // SC_SUBCORES

    @functools.partial(
        pl.kernel,
        out_type=(jax.ShapeDtypeStruct((SC_CORES * npad, 128), F32),
                  jax.ShapeDtypeStruct((SC_CORES * npad, 128), F32)),
        mesh=mesh,
        scratch_types=[
            pltpu.VMEM((SC_CHUNK,), jnp.int32),
            pltpu.VMEM((SC_CHUNK, 128), F32),
            pltpu.VMEM_SHARED((npad, 128), F32),
        ])
    def sk(row_hbm, ef_hbm, aux_hbm, z128_hbm, oagg_hbm, oaux_hbm,
           idx_v, buf_v, sh):
        c = lax.axis_index("c")
        s = lax.axis_index("s")
        wid = c * SC_SUBCORES + s
        nb = pl.multiple_of(s * n_per_sub, 8)
        n_steps = n_per_sub // SC_CHUNK

        def phase(data_hbm, out_hbm):
            pltpu.sync_copy(z128_hbm, buf_v)

            @pl.loop(0, n_steps)
            def _(j):
                o = pl.multiple_of(nb + j * SC_CHUNK, 8)
                pltpu.sync_copy(buf_v, sh.at[pl.ds(o, SC_CHUNK)])

            plsc.subcore_barrier()

            @pl.loop(0, chunks_per_w)
            def _(j):
                ch = wid * chunks_per_w + j
                off_i = pl.multiple_of(ch * SC_CHUNK, SC_CHUNK)
                off_d = pl.multiple_of(
                    jnp.where(ch < data_chunks, ch, 0) * SC_CHUNK, SC_CHUNK)
                pltpu.sync_copy(row_hbm.at[pl.ds(off_i, SC_CHUNK)], idx_v)
                pltpu.sync_copy(data_hbm.at[pl.ds(off_d, SC_CHUNK)], buf_v)
                pltpu.sync_copy(buf_v, sh.at[idx_v], add=True)

            plsc.subcore_barrier()

            @pl.loop(0, n_steps)
            def _(j):
                o = pl.multiple_of(nb + j * SC_CHUNK, 8)
                oo = pl.multiple_of(c * npad + o, 8)
                pltpu.sync_copy(sh.at[pl.ds(o, SC_CHUNK)], buf_v)
                pltpu.sync_copy(buf_v, out_hbm.at[pl.ds(oo, SC_CHUNK)])

        phase(ef_hbm, oagg_hbm)
        phase(aux_hbm, oaux_hbm)

    return sk(rowp, ef, aux, z128)


def _k5_body(h_ref, cpad_ref, agg0_ref, agg1_ref, aux0_ref, aux1_ref,
             wn1a_ref, wn1b_ref, bn1_ref, wn2_ref, bn2_ref,
             hout_ref, cout_ref):
    agg = agg0_ref[0] + agg1_ref[0]
    aux = aux0_ref[0] + aux1_ref[0]
    seg = aux[:, 0:3]
    cnt = aux[:, 3:4]
    cout_ref[...] = cpad_ref[...][:, 0:3] + seg / jnp.maximum(cnt, 1.0)
    hb = h_ref[...]
    t = _silu(jnp.dot(hb, wn1a_ref[...], preferred_element_type=F32)
              + jnp.dot(agg, wn1b_ref[...], preferred_element_type=F32)
              + bn1_ref[...])
    hout_ref[...] = hb + jnp.dot(t, wn2_ref[...], preferred_element_type=F32) \
        + bn2_ref[...]


def _node_update(h, coord_pad, aggp, auxp, wn1a, wn1b, bn1, wn2, bn2):
    n, d = h.shape
    return pl.pallas_call(
        _k5_body,
        grid=(n // BN,),
        in_specs=[
            pl.BlockSpec((BN, d), lambda i: (i, 0)),
            pl.BlockSpec((BN, 128), lambda i: (i, 0)),
            pl.BlockSpec((1, BN, 128), lambda i: (0, i, 0)),
            pl.BlockSpec((1, BN, 128), lambda i: (1, i, 0)),
            pl.BlockSpec((1, BN, 128), lambda i: (0, i, 0)),
            pl.BlockSpec((1, BN, 128), lambda i: (1, i, 0)),
            pl.BlockSpec((d, 128), lambda i: (0, 0)),
            pl.BlockSpec((128, 128), lambda i: (0, 0)),
            pl.BlockSpec((1, 128), lambda i: (0, 0)),
            pl.BlockSpec((128, d), lambda i: (0, 0)),
            pl.BlockSpec((1, d), lambda i: (0, 0)),
        ],
        out_specs=[
            pl.BlockSpec((BN, d), lambda i: (i, 0)),
            pl.BlockSpec((BN, 3), lambda i: (i, 0)),
        ],
        out_shape=[
            jax.ShapeDtypeStruct((n, d), F32),
            jax.ShapeDtypeStruct((n, 3), F32),
        ],
    )(h, coord_pad, aggp, aggp, auxp, auxp, wn1a, wn1b, bn1, wn2, bn2)


def kernel(h, edge_index, coord, edge_attr,
           W_e1, b_e1, W_e2, b_e2,
           W_n1, b_n1, W_n2, b_n2,
           W_c1, b_c1, W_c2):
    n, d = h.shape
    e = edge_index.shape[1]
    row = edge_index[0]
    col = edge_index[1]

    coord_pad = jnp.pad(coord, ((0, 0), (0, 125)))
    w1s = W_e1[:d]
    w1d = W_e1[d:2 * d]
    wr = W_e1[2 * d:2 * d + 1]
    wea = W_e1[2 * d + 1:]
    be1 = b_e1.reshape(1, -1)
    be2 = b_e2.reshape(1, -1)
    bc1 = b_c1.reshape(1, -1)
    bn1 = b_n1.reshape(1, -1)
    bn2 = b_n2.reshape(1, -1)
    wc2t = W_c2.reshape(1, -1)
    wn1a = W_n1[:d]
    wn1b = W_n1[d:]

    tsrc, tdst = _build_tables(h, coord_pad, w1s, w1d, be1)
    epad = -(-e // (GW * SC_WORKERS)) * (GW * SC_WORKERS)
    rowp = jnp.pad(row, (0, epad - e)).reshape(1, epad)
    colp = jnp.pad(col, (0, epad - e)).reshape(1, epad)
    gs = _sc_gather(tsrc, rowp, epad)
    gd = _sc_gather(tdst, colp, epad)
    ef, aux = _edge_compute(gs, gd, edge_attr, wea, wr, we2=W_e2,
                            be2=be2, wc1=W_c1, bc1=bc1, wc2t=wc2t)
    z128 = jnp.zeros((SC_CHUNK, 128), F32)
    row_scat = jnp.concatenate([row, jnp.full((epad - e,), n, jnp.int32)])
    aggp, auxp = _sc_scatter(row_scat, ef, aux, z128, NPAD, e, epad)
    aggp = aggp.reshape(SC_CORES, NPAD, 128)
    auxp = auxp.reshape(SC_CORES, NPAD, 128)
    h_out, coord_out = _node_update(h, coord_pad, aggp, auxp,
                                    wn1a, wn1b, bn1, W_n2, bn2)
    return (h_out, coord_out, edge_attr, ef)
